# Initial kernel scaffold; baseline (speedup 1.0000x reference)
import jax
import jax.numpy as jnp
from jax import lax
from jax.experimental import pallas as pl
from jax.experimental.pallas import tpu as pltpu
from jax.experimental.pallas import tpu_sc as plsc

N = 10000
E = 160000
D = 128
DP = 144
B = 1000
NB = N // B
C = 128
NT = 16
NCHUNK = E // C
RPT = N // NT


def _prep_body(src_a, src_t, dstf, wsw, alw, wdw, arw, wsh, alh, wdh, arh,
               hxw_ref, hxh_ref, elw_ref, erw_ref, elh_ref, erh_ref):
    pad = jnp.where(lax.broadcasted_iota(jnp.int32, (B, 16), 1) == 0, 1.0, 0.0)
    hw = jnp.dot(src_a[...], wsw[...], preferred_element_type=jnp.float32)
    hxw_ref[:, :D] = hw
    hxw_ref[:, D:DP] = pad
    elw_ref[...] = jnp.dot(hw, alw[...], preferred_element_type=jnp.float32)
    hh = jnp.dot(src_t[...], wsh[...], preferred_element_type=jnp.float32)
    hxh_ref[:, :D] = hh
    hxh_ref[:, D:DP] = pad
    elh_ref[...] = jnp.dot(hh, alh[...], preferred_element_type=jnp.float32)
    hdw = jnp.dot(dstf[...], wdw[...], preferred_element_type=jnp.float32)
    erw_ref[...] = jnp.dot(hdw, arw[...], preferred_element_type=jnp.float32)
    hdh = jnp.dot(dstf[...], wdh[...], preferred_element_type=jnp.float32)
    erh_ref[...] = jnp.dot(hdh, arh[...], preferred_element_type=jnp.float32)


def _prep(src_a, src_t, dstf, wsw, alw, wdw, arw, wsh, alh, wdh, arh):
    row = pl.BlockSpec((B, D), lambda i: (i, 0))
    full = pl.BlockSpec((D, D), lambda i: (0, 0))
    vec = pl.BlockSpec((D, 1), lambda i: (0, 0))
    out_row = pl.BlockSpec((B, DP), lambda i: (i, 0))
    out_col = pl.BlockSpec((B, 1), lambda i: (i, 0))
    f32 = jnp.float32
    return pl.pallas_call(
        _prep_body,
        grid=(NB,),
        in_specs=[row, row, row, full, vec, full, vec, full, vec, full, vec],
        out_specs=[out_row, out_row, out_col, out_col, out_col, out_col],
        out_shape=[
            jax.ShapeDtypeStruct((N, DP), f32),
            jax.ShapeDtypeStruct((N, DP), f32),
            jax.ShapeDtypeStruct((N, 1), f32),
            jax.ShapeDtypeStruct((N, 1), f32),
            jax.ShapeDtypeStruct((N, 1), f32),
            jax.ShapeDtypeStruct((N, 1), f32),
        ],
    )(src_a, src_t, dstf, wsw, alw, wdw, arw, wsh, alh, wdh, arh)


def _sc_body(hxw, hxh, elw, erw, elh, erh, sw, dw, sh, dh, zrows, acc,
             el_v, er_v, s_v, d_v, w_v, rows_v, sem, acc_sh):
    cid = lax.axis_index("c")
    sid = lax.axis_index("s")

    pltpu.sync_copy(zrows, acc_sh.at[pl.ds(sid * RPT, RPT)])
    plsc.subcore_barrier()

    def run_rel(el_hbm, er_hbm, s_hbm, d_hbm, hx_hbm, rel):
        pltpu.sync_copy(el_hbm, el_v)
        pltpu.sync_copy(er_hbm, er_v)

        nk = (NCHUNK - sid + NT - 1) // NT

        def chunk(i, _):
            base = (sid + i * NT) * C
            pltpu.sync_copy(s_hbm.at[pl.ds(base, C)], s_v)
            pltpu.sync_copy(d_hbm.at[pl.ds(base, C)], d_v)
            cp = pltpu.async_copy(hx_hbm.at[s_v], rows_v, sem)

            def logits(j, _):
                s16 = s_v[pl.ds(j * 16, 16)]
                d16 = d_v[pl.ds(j * 16, 16)]
                el16 = plsc.load_gather(el_v, [s16])
                er16 = plsc.load_gather(er_v, [d16])
                x = el16 + er16
                w_v[pl.ds(j * 16, 16)] = jnp.exp(
                    jnp.where(x >= 0.0, x, 0.2 * x))
                return 0

            lax.fori_loop(0, C // 16, logits, 0)
            cp.wait()

            def scale(r, _):
                wr = plsc.load_gather(w_v, [jnp.broadcast_to(r, (16,))])
                for j in range(DP // 16):
                    rows_v[r, pl.ds(j * 16, 16)] = (
                        rows_v[r, pl.ds(j * 16, 16)] * wr)
                return 0

            lax.fori_loop(0, C, scale, 0)
            pltpu.sync_copy(rows_v, acc_sh.at[d_v], add=True)
            return 0

        lax.fori_loop(0, nk, chunk, 0)
        plsc.subcore_barrier()
        pltpu.sync_copy(acc_sh.at[pl.ds(sid * RPT, RPT)],
                        acc.at[rel, pl.ds(sid * RPT, RPT)])

    @pl.when(cid == 0)
    def _():
        run_rel(elw, erw, sw, dw, hxw, 0)

    @pl.when(cid == 1)
    def _():
        run_rel(elh, erh, sh, dh, hxh, 1)


def _sc_aggregate(hxw, hxh, elw, erw, elh, erh, sw, dw, sh, dh):
    f32 = jnp.float32
    zrows = jnp.zeros((RPT, DP), f32)
    mesh = plsc.VectorSubcoreMesh(core_axis_name="c", subcore_axis_name="s")
    return pl.kernel(
        _sc_body,
        out_type=jax.ShapeDtypeStruct((2, N, DP), f32),
        mesh=mesh,
        scratch_types=[
            pltpu.VMEM((N,), f32),
            pltpu.VMEM((N,), f32),
            pltpu.VMEM((C,), jnp.int32),
            pltpu.VMEM((C,), jnp.int32),
            pltpu.VMEM((C,), f32),
            pltpu.VMEM((C, DP), f32),
            pltpu.SemaphoreType.DMA,
            pltpu.VMEM_SHARED((N, DP), f32),
        ],
    )(hxw, hxh, elw, erw, elh, erh, sw, dw, sh, dh, zrows)


def _epi_a_body(acc0, acc1, bw, bh, w1, b1, w2, z0_ref, z1_ref, part_ref):
    def one(a_ref, b_ref):
        h = a_ref[:, :D]
        den = a_ref[:, D:D + 1]
        x = h / (den + 1e-9) + b_ref[...]
        z = jnp.where(x > 0.0, x, jnp.expm1(x))
        t = jnp.tanh(jnp.dot(z, w1[...], preferred_element_type=jnp.float32)
                     + b1[...])
        s = jnp.sum(jnp.dot(t, w2[...], preferred_element_type=jnp.float32))
        return z, s

    z0, s0 = one(acc0, bw)
    z1, s1 = one(acc1, bh)
    z0_ref[...] = z0
    z1_ref[...] = z1
    ii = lax.broadcasted_iota(jnp.int32, (1, D), 1)
    part_ref[...] = jnp.where(ii == 0, s0, jnp.where(ii == 1, s1, 0.0))


def _epi_a(acc0, acc1, bw, bh, w1, b1, w2):
    f32 = jnp.float32
    arow = pl.BlockSpec((B, DP), lambda i: (i, 0))
    brow = pl.BlockSpec((1, D), lambda i: (0, 0))
    full = pl.BlockSpec((D, D), lambda i: (0, 0))
    vec = pl.BlockSpec((D, 1), lambda i: (0, 0))
    zrow = pl.BlockSpec((B, D), lambda i: (i, 0))
    prow = pl.BlockSpec((1, D), lambda i: (i, 0))
    return pl.pallas_call(
        _epi_a_body,
        grid=(NB,),
        in_specs=[arow, arow, brow, brow, full, brow, vec],
        out_specs=[zrow, zrow, prow],
        out_shape=[
            jax.ShapeDtypeStruct((N, D), f32),
            jax.ShapeDtypeStruct((N, D), f32),
            jax.ShapeDtypeStruct((NB, D), f32),
        ],
    )(acc0, acc1, bw, bh, w1, b1, w2)


def _epi_b_body(z0, z1, part, z_ref, att_ref):
    s0 = jnp.sum(part[:, 0:1]) / N
    s1 = jnp.sum(part[:, 1:2]) / N
    m = jnp.maximum(s0, s1)
    e0 = jnp.exp(s0 - m)
    e1 = jnp.exp(s1 - m)
    a0 = e0 / (e0 + e1)
    a1 = e1 / (e0 + e1)
    z_ref[...] = a0 * z0[...] + a1 * z1[...]
    ii = lax.broadcasted_iota(jnp.int32, (1, D), 1)
    att_ref[...] = jnp.where(ii == 0, a0, jnp.where(ii == 1, a1, 0.0))


def _epi_b(z0, z1, part):
    f32 = jnp.float32
    zrow = pl.BlockSpec((B, D), lambda i: (i, 0))
    pfull = pl.BlockSpec((NB, D), lambda i: (0, 0))
    afull = pl.BlockSpec((1, D), lambda i: (0, 0))
    return pl.pallas_call(
        _epi_b_body,
        grid=(NB,),
        in_specs=[zrow, zrow, pfull],
        out_specs=[zrow, afull],
        out_shape=[
            jax.ShapeDtypeStruct((N, D), f32),
            jax.ShapeDtypeStruct((1, D), f32),
        ],
    )(z0, z1, part)


def kernel(dst_feat, src_feat_author, src_feat_term, edge_index_writes,
           edge_index_has, Wsrc_writes, Wdst_writes, al_writes, ar_writes,
           bias_writes, Wsrc_has, Wdst_has, al_has, ar_has, bias_has,
           W1_sem, b1_sem, w2_sem):
    hxw, hxh, elw, erw, elh, erh = _prep(
        src_feat_author, src_feat_term, dst_feat,
        Wsrc_writes, al_writes.reshape(D, 1),
        Wdst_writes, ar_writes.reshape(D, 1),
        Wsrc_has, al_has.reshape(D, 1),
        Wdst_has, ar_has.reshape(D, 1))

    acc = _sc_aggregate(
        hxw, hxh,
        elw.reshape(N), erw.reshape(N), elh.reshape(N), erh.reshape(N),
        edge_index_writes[0], edge_index_writes[1],
        edge_index_has[0], edge_index_has[1])

    z0, z1, part = _epi_a(
        acc[0], acc[1],
        bias_writes.reshape(1, D), bias_has.reshape(1, D),
        W1_sem, b1_sem.reshape(1, D), w2_sem)

    z, att = _epi_b(z0, z1, part)
    return (z, att[0, :2])


# speedup vs baseline: 18.6485x; 18.6485x over previous
import jax
import jax.numpy as jnp
from jax import lax
from jax.experimental import pallas as pl
from jax.experimental.pallas import tpu as pltpu
from jax.experimental.pallas import tpu_sc as plsc

N = 10000
E = 160000
D = 128
DP = 144
B = 1000
NB = N // B
C = 128
NT = 16
NCHUNK = E // C
NP = 10240
RPT = NP // NT


def _prep_body(src_a, src_t, dstf, wsw, alw, wdw, arw, wsh, alh, wdh, arh,
               hxw_ref, hxh_ref, elw_ref, erw_ref, elh_ref, erh_ref):
    pad = jnp.where(lax.broadcasted_iota(jnp.int32, (B, 16), 1) == 0, 1.0, 0.0)
    hw = jnp.dot(src_a[...], wsw[...], preferred_element_type=jnp.float32)
    hxw_ref[:, :D] = hw
    hxw_ref[:, D:DP] = pad
    elw_ref[...] = jnp.dot(hw, alw[...], preferred_element_type=jnp.float32)
    hh = jnp.dot(src_t[...], wsh[...], preferred_element_type=jnp.float32)
    hxh_ref[:, :D] = hh
    hxh_ref[:, D:DP] = pad
    elh_ref[...] = jnp.dot(hh, alh[...], preferred_element_type=jnp.float32)
    hdw = jnp.dot(dstf[...], wdw[...], preferred_element_type=jnp.float32)
    erw_ref[...] = jnp.dot(hdw, arw[...], preferred_element_type=jnp.float32)
    hdh = jnp.dot(dstf[...], wdh[...], preferred_element_type=jnp.float32)
    erh_ref[...] = jnp.dot(hdh, arh[...], preferred_element_type=jnp.float32)


def _prep(src_a, src_t, dstf, wsw, alw, wdw, arw, wsh, alh, wdh, arh):
    row = pl.BlockSpec((B, D), lambda i: (i, 0))
    full = pl.BlockSpec((D, D), lambda i: (0, 0))
    vec = pl.BlockSpec((D, 1), lambda i: (0, 0))
    out_row = pl.BlockSpec((B, DP), lambda i: (i, 0))
    out_col = pl.BlockSpec((B, 1), lambda i: (i, 0))
    f32 = jnp.float32
    return pl.pallas_call(
        _prep_body,
        grid=(NB,),
        in_specs=[row, row, row, full, vec, full, vec, full, vec, full, vec],
        out_specs=[out_row, out_row, out_col, out_col, out_col, out_col],
        out_shape=[
            jax.ShapeDtypeStruct((N, DP), f32),
            jax.ShapeDtypeStruct((N, DP), f32),
            jax.ShapeDtypeStruct((N, 1), f32),
            jax.ShapeDtypeStruct((N, 1), f32),
            jax.ShapeDtypeStruct((N, 1), f32),
            jax.ShapeDtypeStruct((N, 1), f32),
        ],
    )(src_a, src_t, dstf, wsw, alw, wdw, arw, wsh, alh, wdh, arh)


def _sc_body(hxw, hxh, elw, erw, elh, erh, sw, dw, sh, dh, zrows, acc,
             el_v, er_v, s_v, d_v, w_v, rows_v, sem, acc_sh):
    cid = lax.axis_index("c")
    sid = lax.axis_index("s")

    pltpu.sync_copy(zrows, acc_sh.at[pl.ds(sid * RPT, RPT)])
    plsc.subcore_barrier()

    def run_rel(el_hbm, er_hbm, s_hbm, d_hbm, hx_hbm, rel):
        pltpu.sync_copy(el_hbm, el_v)
        pltpu.sync_copy(er_hbm, er_v)

        nk = (NCHUNK - sid + NT - 1) // NT

        def chunk(i, _):
            base = (sid + i * NT) * C
            pltpu.sync_copy(s_hbm.at[pl.ds(base, C)], s_v)
            pltpu.sync_copy(d_hbm.at[pl.ds(base, C)], d_v)
            cp = pltpu.async_copy(hx_hbm.at[s_v], rows_v, sem)

            def logits(j, _):
                s16 = s_v[pl.ds(j * 16, 16)]
                d16 = d_v[pl.ds(j * 16, 16)]
                el16 = plsc.load_gather(el_v, [s16])
                er16 = plsc.load_gather(er_v, [d16])
                x = el16 + er16
                w_v[pl.ds(j * 16, 16)] = jnp.exp(
                    jnp.where(x >= 0.0, x, 0.2 * x))
                return 0

            lax.fori_loop(0, C // 16, logits, 0)
            cp.wait()

            def scale(r, _):
                wr = plsc.load_gather(w_v, [jnp.broadcast_to(r, (16,))])
                for j in range(DP // 16):
                    rows_v[r, pl.ds(j * 16, 16)] = (
                        rows_v[r, pl.ds(j * 16, 16)] * wr)
                return 0

            lax.fori_loop(0, C, scale, 0)
            pltpu.sync_copy(rows_v, acc_sh.at[d_v], add=True)
            return 0

        lax.fori_loop(0, nk, chunk, 0)
        plsc.subcore_barrier()
        pltpu.sync_copy(acc_sh.at[pl.ds(sid * RPT, RPT)],
                        acc.at[rel, pl.ds(sid * RPT, RPT)])

    @pl.when(cid == 0)
    def _():
        run_rel(elw, erw, sw, dw, hxw, 0)

    @pl.when(cid == 1)
    def _():
        run_rel(elh, erh, sh, dh, hxh, 1)


def _sc_aggregate(hxw, hxh, elw, erw, elh, erh, sw, dw, sh, dh):
    f32 = jnp.float32
    zrows = jnp.zeros((RPT, DP), f32)
    mesh = plsc.VectorSubcoreMesh(core_axis_name="c", subcore_axis_name="s")
    return pl.kernel(
        _sc_body,
        out_type=jax.ShapeDtypeStruct((2, NP, DP), f32),
        mesh=mesh,
        compiler_params=pltpu.CompilerParams(needs_layout_passes=False,
                                             use_tc_tiling_on_sc=False),
        scratch_types=[
            pltpu.VMEM((N,), f32),
            pltpu.VMEM((N,), f32),
            pltpu.VMEM((C,), jnp.int32),
            pltpu.VMEM((C,), jnp.int32),
            pltpu.VMEM((C,), f32),
            pltpu.VMEM((C, DP), f32),
            pltpu.SemaphoreType.DMA,
            pltpu.VMEM_SHARED((NP, DP), f32),
        ],
    )(hxw, hxh, elw, erw, elh, erh, sw, dw, sh, dh, zrows)


def _epi_a_body(acc0, acc1, bw, bh, w1, b1, w2, z0_ref, z1_ref, part_ref):
    def one(a_ref, b_ref):
        h = a_ref[:, :D]
        den = a_ref[:, D:D + 1]
        x = h / (den + 1e-9) + b_ref[...]
        z = jnp.where(x > 0.0, x, jnp.exp(jnp.minimum(x, 0.0)) - 1.0)
        t = jnp.tanh(jnp.dot(z, w1[...], preferred_element_type=jnp.float32)
                     + b1[...])
        s = jnp.sum(jnp.dot(t, w2[...], preferred_element_type=jnp.float32))
        return z, s

    z0, s0 = one(acc0, bw)
    z1, s1 = one(acc1, bh)
    z0_ref[...] = z0
    z1_ref[...] = z1
    ii = lax.broadcasted_iota(jnp.int32, (1, 8, D), 2)
    part_ref[...] = jnp.where(ii == 0, s0, jnp.where(ii == 1, s1, 0.0))


def _epi_a(acc0, acc1, bw, bh, w1, b1, w2):
    f32 = jnp.float32
    arow = pl.BlockSpec((B, DP), lambda i: (i, 0))
    brow = pl.BlockSpec((1, D), lambda i: (0, 0))
    full = pl.BlockSpec((D, D), lambda i: (0, 0))
    vec = pl.BlockSpec((D, 1), lambda i: (0, 0))
    zrow = pl.BlockSpec((B, D), lambda i: (i, 0))
    prow = pl.BlockSpec((1, 8, D), lambda i: (i, 0, 0))
    return pl.pallas_call(
        _epi_a_body,
        grid=(NB,),
        in_specs=[arow, arow, brow, brow, full, brow, vec],
        out_specs=[zrow, zrow, prow],
        out_shape=[
            jax.ShapeDtypeStruct((N, D), f32),
            jax.ShapeDtypeStruct((N, D), f32),
            jax.ShapeDtypeStruct((NB, 8, D), f32),
        ],
    )(acc0, acc1, bw, bh, w1, b1, w2)


def _epi_b_body(z0, z1, part, z_ref, att_ref):
    s0 = jnp.sum(part[:, 0, 0:1]) / N
    s1 = jnp.sum(part[:, 0, 1:2]) / N
    m = jnp.maximum(s0, s1)
    e0 = jnp.exp(s0 - m)
    e1 = jnp.exp(s1 - m)
    a0 = e0 / (e0 + e1)
    a1 = e1 / (e0 + e1)
    z_ref[...] = a0 * z0[...] + a1 * z1[...]
    ii = lax.broadcasted_iota(jnp.int32, (1, D), 1)
    att_ref[...] = jnp.where(ii == 0, a0, jnp.where(ii == 1, a1, 0.0))


def _epi_b(z0, z1, part):
    f32 = jnp.float32
    zrow = pl.BlockSpec((B, D), lambda i: (i, 0))
    pfull = pl.BlockSpec((NB, 8, D), lambda i: (0, 0, 0))
    afull = pl.BlockSpec((1, D), lambda i: (0, 0))
    return pl.pallas_call(
        _epi_b_body,
        grid=(NB,),
        in_specs=[zrow, zrow, pfull],
        out_specs=[zrow, afull],
        out_shape=[
            jax.ShapeDtypeStruct((N, D), f32),
            jax.ShapeDtypeStruct((1, D), f32),
        ],
    )(z0, z1, part)


def kernel(dst_feat, src_feat_author, src_feat_term, edge_index_writes,
           edge_index_has, Wsrc_writes, Wdst_writes, al_writes, ar_writes,
           bias_writes, Wsrc_has, Wdst_has, al_has, ar_has, bias_has,
           W1_sem, b1_sem, w2_sem):
    hxw, hxh, elw, erw, elh, erh = _prep(
        src_feat_author, src_feat_term, dst_feat,
        Wsrc_writes, al_writes.reshape(D, 1),
        Wdst_writes, ar_writes.reshape(D, 1),
        Wsrc_has, al_has.reshape(D, 1),
        Wdst_has, ar_has.reshape(D, 1))

    acc = _sc_aggregate(
        hxw, hxh,
        elw.reshape(N), erw.reshape(N), elh.reshape(N), erh.reshape(N),
        edge_index_writes[0], edge_index_writes[1],
        edge_index_has[0], edge_index_has[1])

    z0, z1, part = _epi_a(
        acc[0, :N], acc[1, :N],
        bias_writes.reshape(1, D), bias_has.reshape(1, D),
        W1_sem, b1_sem.reshape(1, D), w2_sem)

    z, att = _epi_b(z0, z1, part)
    return (z, att[0, :2])
